# baseline (speedup 1.0000x reference)
import functools

import jax
import jax.numpy as jnp
from jax import lax
from jax.experimental import pallas as pl
from jax.experimental.pallas import tpu as pltpu, tpu_sc as plsc

M = 50000
K = 25000
N_INC = 320000
NC, NS, L = 2, 16, 16
FULL = 50176
TPC = FULL // NS
CHUNK = 1568
NVREG = CHUNK // L
VB = 16
HALF = FULL // 2
HLOC = HALF + L
EPT2 = N_INC // (NS // 2)
EPTH = EPT2 // 2
SOFF = NS * HLOC
TOFF = SOFF + 32 * VB
TAIL = M - 31 * CHUNK
PADV = float((NVREG - TAIL // L) * L)

_mesh = plsc.VectorSubcoreMesh(
    core_axis_name="c", subcore_axis_name="s", num_cores=NC, num_subcores=NS)


@functools.partial(
    pl.kernel,
    out_type=(
        jax.ShapeDtypeStruct((M,), jnp.float32),
        jax.ShapeDtypeStruct((M,), jnp.float32),
        jax.ShapeDtypeStruct((M,), jnp.float32),
    ),
    mesh=_mesh,
    compiler_params=pltpu.CompilerParams(needs_layout_passes=False),
    scratch_types=(
        pltpu.VMEM((EPTH,), jnp.float32),
        pltpu.VMEM((HLOC,), jnp.float32),
        pltpu.VMEM_SHARED((SOFF + 64 * VB,), jnp.float32),
        pltpu.VMEM((TPC,), jnp.float32),
        pltpu.VMEM((TPC,), jnp.float32),
        pltpu.VMEM((TPC,), jnp.float32),
        pltpu.VMEM((VB,), jnp.float32),
        pltpu.VMEM((L,), jnp.float32),
        pltpu.VMEM((64 * VB,), jnp.float32),
        pltpu.VMEM((3 * CHUNK,), jnp.float32),
        pltpu.VMEM(((NVREG + 14) * L,), jnp.float32),
        pltpu.VMEM((NVREG + 14,), jnp.float32),
        pltpu.SemaphoreType.DMA,
        pltpu.SemaphoreType.DMA,
        pltpu.SemaphoreType.DMA,
        pltpu.SemaphoreType.DMA,
        pltpu.SemaphoreType.DMA,
        pltpu.SemaphoreType.DMA,
        pltpu.SemaphoreType.DMA,
    ),
)
def _degree_mask(e_hbm, probs_hbm, soft_hbm, hard_hbm,
                 idx_v, hist_v, stage_sh, acc_v, tmp_a, tmp_b, h16_v, spl_v,
                 stats_v, out_v, cs_v, off_v,
                 sem0, s1, s2, s3, s4, s5, s6):
    c = lax.axis_index("c")
    s = lax.axis_index("s")
    zeros = jnp.zeros((L,), jnp.float32)
    ones = jnp.ones((L,), jnp.float32)

    hh = s // (NS // 2)
    es = s % (NS // 2)
    half_base = hh * HALF
    stage = pltpu.async_copy(e_hbm.at[pl.ds(es * EPT2, EPTH)], idx_v, sem0)

    def _z(i, _):
        hist_v[pl.ds(i * L, L)] = zeros
        return 0
    lax.fori_loop(0, HLOC // L, _z, 0, unroll=8)

    SU = 5
    def _scat(j, _):
        vs = [idx_v[pl.ds((j * SU + u) * L, L)] for u in range(SU)]
        sels = []
        for v in vs:
            local = plsc.bitcast(v, jnp.int32) - half_base
            inb = (local >= 0) & (local < HALF)
            sels.append(jnp.where(inb, local, HALF))
        for sel in sels:
            plsc.addupdate_scatter(hist_v, [sel], ones)
        return 0
    for p in range(2):
        stage.wait()
        if p == 0:
            lax.fori_loop(0, EPTH // L // SU, _scat, 0)
            stage = pltpu.async_copy(
                e_hbm.at[pl.ds(es * EPT2 + EPTH, EPTH)], idx_v, sem0)
        else:
            lax.fori_loop(0, EPTH // L // SU, _scat, 0)

    pltpu.sync_copy(hist_v, stage_sh.at[pl.ds(s * HLOC, HLOC)])
    plsc.subcore_barrier()

    loff = (s % (NS // 2)) * TPC
    rbufs = (tmp_a, tmp_b,
             idx_v.at[pl.ds(0, TPC)], idx_v.at[pl.ds(TPC, TPC)],
             idx_v.at[pl.ds(2 * TPC, TPC)], out_v.at[pl.ds(0, TPC)])
    rsems = (s1, s2, s3, s4, s5, s6)
    descs = [None] * 8
    def _fetch(p):
        b = rbufs[p % 6]
        descs[p] = pltpu.async_copy(
            stage_sh.at[pl.ds((hh * 8 + p) * HLOC + loff, TPC)], b,
            rsems[p % 6])
    for p in range(6):
        _fetch(p)
    passes = [[0, 1, 2, 3], [4, 5, 6, 7]]
    AU = 7
    for pi, rows in enumerate(passes):
        for p in rows:
            descs[p].wait()
        bs = [rbufs[p % 6] for p in rows]
        first = (pi == 0)
        def _add(j, _, bs=bs, first=first):
            for u in range(AU):
                sl = pl.ds((j * AU + u) * L, L)
                v = bs[0][sl] if first else acc_v[sl] + bs[0][sl]
                for b in bs[1:]:
                    v = v + b[sl]
                acc_v[sl] = v
            return 0
        lax.fori_loop(0, TPC // L // AU, _add, 0)
        if pi == 0:
            for p in (6, 7):
                _fetch(p)

    iota = lax.iota(jnp.int32, VB)
    for h in range(2):
        q2 = 2 * s + h
        h16_v[...] = zeros
        def _stat(g, sv, h=h):
            dvs = [acc_v[pl.ds(h * CHUNK + (g * AU + u) * L, L)]
                   for u in range(AU)]
            cis = [jnp.minimum(dv, float(VB - 1)).astype(jnp.int32)
                   for dv in dvs]
            for ci in cis:
                plsc.addupdate_scatter(h16_v, [ci], ones)
            for dv in dvs:
                sv = sv + dv
            return sv
        sv = lax.fori_loop(0, NVREG // AU, _stat, zeros)

        @pl.when(q2 == 31)
        def _():
            h16_v[...] = h16_v[...] - jnp.where(iota == 0, PADV, 0.0)
        pltpu.sync_copy(h16_v, stage_sh.at[pl.ds(SOFF + q2 * VB, VB)])
        spl_v[...] = jnp.full((L,), jnp.sum(sv), jnp.float32)
        pltpu.sync_copy(spl_v, stage_sh.at[pl.ds(TOFF + q2 * VB, VB)])
    plsc.subcore_barrier()

    pltpu.sync_copy(stage_sh.at[pl.ds(SOFF, 64 * VB)], stats_v)

    def _acc2(v, carry):
        hv, tv = carry
        return (hv + stats_v[pl.ds(v * VB, VB)],
                tv + stats_v[pl.ds(32 * VB + v * VB, VB)])
    hvec, tvec = lax.fori_loop(
        0, 32, _acc2,
        (jnp.zeros((VB,), jnp.float32), jnp.zeros((L,), jnp.float32)),
        unroll=8)

    c_ge = jnp.flip(jnp.cumsum(jnp.flip(hvec, 0)), 0)
    ge_mask = c_ge >= float(K)
    t_i = plsc.all_reduce_population_count(ge_mask) - 1
    t_f32 = t_i.astype(jnp.float32)
    c_gt = jnp.sum(jnp.where(iota > t_i, hvec, 0.0))
    r = float(K) - c_gt

    q = 2 * s + c
    def _off(v, off):
        tie_v = jnp.sum(jnp.where(iota == t_i, stats_v[pl.ds(v * VB, VB)], 0.0))
        return off + jnp.where(v < q, tie_v, 0.0)
    offset = lax.fori_loop(0, 32, _off, jnp.float32(0.0), unroll=8)

    cb = c * CHUNK
    def _pa(g, _):
        for u in range(AU):
            j = g * AU + u
            dv = acc_v[pl.ds(cb + j * L, L)]
            eqf = (dv == t_f32).astype(jnp.float32)
            out_v[pl.ds(j * L, L)] = dv / tvec
            cs_v[pl.ds(j * L, L)] = jnp.cumsum(eqf)
        return 0
    lax.fori_loop(0, NVREG // AU, _pa, 0)

    lane15 = iota * L + L - 1
    def _pb(u, carry):
        cvec = plsc.load_gather(cs_v, [lane15 + u * (L * L)])
        csum = jnp.cumsum(cvec)
        off_v[pl.ds(u * L, L)] = csum - cvec + carry
        return carry + jnp.sum(cvec)
    lax.fori_loop(0, (NVREG + L - 1) // L, _pb, offset)

    def _pc(g, _):
        for u in range(AU):
            j = g * AU + u
            ofs = plsc.load_gather(off_v, [jnp.zeros((L,), jnp.int32) + j])
            crank = cs_v[pl.ds(j * L, L)] + ofs
            dv = acc_v[pl.ds(cb + j * L, L)]
            eq = dv == t_f32
            keep = (dv > t_f32) | (eq & (crank <= r))
            hv = jnp.where(keep, 1.0, 0.0)
            pv = out_v[pl.ds(j * L, L)]
            out_v[pl.ds(CHUNK + j * L, L)] = (hv - pv) + pv
            out_v[pl.ds(2 * CHUNK + j * L, L)] = hv
        return 0
    lax.fori_loop(0, NVREG // AU, _pc, 0)

    base = q * CHUNK

    @pl.when(q < 31)
    def _():
        pltpu.sync_copy(out_v.at[pl.ds(0, CHUNK)],
                        probs_hbm.at[pl.ds(base, CHUNK)])
        pltpu.sync_copy(out_v.at[pl.ds(CHUNK, CHUNK)],
                        soft_hbm.at[pl.ds(base, CHUNK)])
        pltpu.sync_copy(out_v.at[pl.ds(2 * CHUNK, CHUNK)],
                        hard_hbm.at[pl.ds(base, CHUNK)])

    @pl.when(q == 31)
    def _():
        pltpu.sync_copy(out_v.at[pl.ds(0, TAIL)],
                        probs_hbm.at[pl.ds(base, TAIL)])
        pltpu.sync_copy(out_v.at[pl.ds(CHUNK, TAIL)],
                        soft_hbm.at[pl.ds(base, TAIL)])
        pltpu.sync_copy(out_v.at[pl.ds(2 * CHUNK, TAIL)],
                        hard_hbm.at[pl.ds(base, TAIL)])


def kernel(x, V_idx, E_idx, num_nodes, num_hyperedges, token_valid, inv_node,
           is_test):
    e = lax.bitcast_convert_type(E_idx.astype(jnp.int32), jnp.float32)
    return _degree_mask(e)


# speedup vs baseline: 1.5340x; 1.5340x over previous
import functools

import jax
import jax.numpy as jnp
from jax import lax
from jax.experimental import pallas as pl
from jax.experimental.pallas import tpu as pltpu, tpu_sc as plsc

M = 50000
K = 25000
N_INC = 320000
NC, NS, L = 2, 16, 16
FULL = 50176
TPC = FULL // NS
CHUNK = 1568
NVREG = CHUNK // L
EPT = N_INC // NS
EPTH = EPT // 2
VB = 16
SOFF = NS * FULL
TOFF = SOFF + 32 * VB
TAIL = M - 31 * CHUNK
PADV = float((NVREG - TAIL // L) * L)

_mesh = plsc.VectorSubcoreMesh(
    core_axis_name="c", subcore_axis_name="s", num_cores=NC, num_subcores=NS)


@functools.partial(
    pl.kernel,
    out_type=(
        jax.ShapeDtypeStruct((M,), jnp.float32),
        jax.ShapeDtypeStruct((M,), jnp.float32),
        jax.ShapeDtypeStruct((M,), jnp.float32),
    ),
    mesh=_mesh,
    compiler_params=pltpu.CompilerParams(needs_layout_passes=False),
    scratch_types=(
        pltpu.VMEM((EPTH,), jnp.float32),
        pltpu.VMEM((FULL,), jnp.float32),
        pltpu.VMEM_SHARED((SOFF + 64 * VB,), jnp.float32),
        pltpu.VMEM((TPC,), jnp.float32),
        pltpu.VMEM((TPC,), jnp.float32),
        pltpu.VMEM((TPC,), jnp.float32),
        pltpu.VMEM((VB,), jnp.float32),
        pltpu.VMEM((L,), jnp.float32),
        pltpu.VMEM((64 * VB,), jnp.float32),
        pltpu.VMEM((3 * CHUNK,), jnp.float32),
        pltpu.VMEM(((NVREG + 14) * L,), jnp.float32),
        pltpu.VMEM((NVREG + 14,), jnp.float32),
        pltpu.SemaphoreType.DMA,
        pltpu.SemaphoreType.DMA,
        pltpu.SemaphoreType.DMA,
        pltpu.SemaphoreType.DMA,
        pltpu.SemaphoreType.DMA,
        pltpu.SemaphoreType.DMA,
        pltpu.SemaphoreType.DMA,
    ),
)
def _degree_mask(e_hbm, probs_hbm, soft_hbm, hard_hbm,
                 idx_v, hist_v, stage_sh, acc_v, tmp_a, tmp_b, h16_v, spl_v,
                 stats_v, out_v, cs_v, off_v,
                 sem0, s1, s2, s3, s4, s5, s6):
    c = lax.axis_index("c")
    s = lax.axis_index("s")
    zeros = jnp.zeros((L,), jnp.float32)
    ones = jnp.ones((L,), jnp.float32)

    stage = pltpu.async_copy(e_hbm.at[pl.ds(s * EPT, EPTH)], idx_v, sem0)

    def _z(i, _):
        hist_v[pl.ds(i * L, L)] = zeros
        return 0
    lax.fori_loop(0, FULL // L, _z, 0, unroll=8)

    SU = 5
    def _scat(j, _):
        vs = [idx_v[pl.ds((j * SU + u) * L, L)] for u in range(SU)]
        for v in vs:
            plsc.addupdate_scatter(hist_v, [plsc.bitcast(v, jnp.int32)], ones)
        return 0
    for p in range(2):
        stage.wait()
        if p == 0:
            lax.fori_loop(0, EPTH // L // SU, _scat, 0)
            stage = pltpu.async_copy(
                e_hbm.at[pl.ds(s * EPT + EPTH, EPTH)], idx_v, sem0)
        else:
            lax.fori_loop(0, EPTH // L // SU, _scat, 0)

    pltpu.sync_copy(hist_v, stage_sh.at[pl.ds(s * FULL, FULL)])
    plsc.subcore_barrier()

    rbufs = (tmp_a, tmp_b,
             idx_v.at[pl.ds(0, TPC)], idx_v.at[pl.ds(TPC, TPC)],
             idx_v.at[pl.ds(2 * TPC, TPC)], out_v.at[pl.ds(0, TPC)])
    rsems = (s1, s2, s3, s4, s5, s6)
    descs = [None] * 16
    def _fetch(row):
        b = rbufs[row % 6]
        descs[row] = pltpu.async_copy(
            stage_sh.at[pl.ds(row * FULL + s * TPC, TPC)], b, rsems[row % 6])
    for row in range(6):
        _fetch(row)
    passes = [[0, 1, 2, 3], [4, 5, 6], [7, 8, 9], [10, 11, 12], [13, 14, 15]]
    AU = 7
    for pi, rows in enumerate(passes):
        for row in rows:
            descs[row].wait()
        bs = [rbufs[row % 6] for row in rows]
        first = (pi == 0)
        def _add(j, _, bs=bs, first=first):
            for u in range(AU):
                sl = pl.ds((j * AU + u) * L, L)
                v = bs[0][sl] if first else acc_v[sl] + bs[0][sl]
                for b in bs[1:]:
                    v = v + b[sl]
                acc_v[sl] = v
            return 0
        lax.fori_loop(0, TPC // L // AU, _add, 0)
        nxt = {0: [6, 7, 8, 9], 1: [10, 11, 12], 2: [13, 14, 15]}.get(pi, [])
        for row in nxt:
            _fetch(row)

    iota = lax.iota(jnp.int32, VB)
    for h in range(2):
        q2 = 2 * s + h
        h16_v[...] = zeros
        def _stat(g, sv, h=h):
            dvs = [acc_v[pl.ds(h * CHUNK + (g * AU + u) * L, L)]
                   for u in range(AU)]
            cis = [jnp.minimum(dv, float(VB - 1)).astype(jnp.int32)
                   for dv in dvs]
            for ci in cis:
                plsc.addupdate_scatter(h16_v, [ci], ones)
            for dv in dvs:
                sv = sv + dv
            return sv
        sv = lax.fori_loop(0, NVREG // AU, _stat, zeros)

        @pl.when(q2 == 31)
        def _():
            h16_v[...] = h16_v[...] - jnp.where(iota == 0, PADV, 0.0)
        pltpu.sync_copy(h16_v, stage_sh.at[pl.ds(SOFF + q2 * VB, VB)])
        spl_v[...] = jnp.full((L,), jnp.sum(sv), jnp.float32)
        pltpu.sync_copy(spl_v, stage_sh.at[pl.ds(TOFF + q2 * VB, VB)])
    plsc.subcore_barrier()

    pltpu.sync_copy(stage_sh.at[pl.ds(SOFF, 64 * VB)], stats_v)

    def _acc2(v, carry):
        hv, tv = carry
        return (hv + stats_v[pl.ds(v * VB, VB)],
                tv + stats_v[pl.ds(32 * VB + v * VB, VB)])
    hvec, tvec = lax.fori_loop(
        0, 32, _acc2,
        (jnp.zeros((VB,), jnp.float32), jnp.zeros((L,), jnp.float32)),
        unroll=8)

    c_ge = jnp.flip(jnp.cumsum(jnp.flip(hvec, 0)), 0)
    ge_mask = c_ge >= float(K)
    t_i = plsc.all_reduce_population_count(ge_mask) - 1
    t_f32 = t_i.astype(jnp.float32)
    c_gt = jnp.sum(jnp.where(iota > t_i, hvec, 0.0))
    r = float(K) - c_gt

    q = 2 * s + c
    def _off(v, off):
        tie_v = jnp.sum(jnp.where(iota == t_i, stats_v[pl.ds(v * VB, VB)], 0.0))
        return off + jnp.where(v < q, tie_v, 0.0)
    offset = lax.fori_loop(0, 32, _off, jnp.float32(0.0), unroll=8)

    cb = c * CHUNK
    def _pa(g, _):
        for u in range(AU):
            j = g * AU + u
            dv = acc_v[pl.ds(cb + j * L, L)]
            eqf = (dv == t_f32).astype(jnp.float32)
            out_v[pl.ds(j * L, L)] = dv / tvec
            cs_v[pl.ds(j * L, L)] = jnp.cumsum(eqf)
        return 0
    lax.fori_loop(0, NVREG // AU, _pa, 0)

    lane15 = iota * L + L - 1
    def _pb(u, carry):
        cvec = plsc.load_gather(cs_v, [lane15 + u * (L * L)])
        csum = jnp.cumsum(cvec)
        off_v[pl.ds(u * L, L)] = csum - cvec + carry
        return carry + jnp.sum(cvec)
    lax.fori_loop(0, (NVREG + L - 1) // L, _pb, offset)

    def _pc(g, _):
        for u in range(AU):
            j = g * AU + u
            ofs = plsc.load_gather(off_v, [jnp.zeros((L,), jnp.int32) + j])
            crank = cs_v[pl.ds(j * L, L)] + ofs
            dv = acc_v[pl.ds(cb + j * L, L)]
            eq = dv == t_f32
            keep = (dv > t_f32) | (eq & (crank <= r))
            hv = jnp.where(keep, 1.0, 0.0)
            pv = out_v[pl.ds(j * L, L)]
            out_v[pl.ds(CHUNK + j * L, L)] = (hv - pv) + pv
            out_v[pl.ds(2 * CHUNK + j * L, L)] = hv
        return 0
    lax.fori_loop(0, NVREG // AU, _pc, 0)

    base = q * CHUNK

    @pl.when(q < 31)
    def _():
        pltpu.sync_copy(out_v.at[pl.ds(0, CHUNK)],
                        probs_hbm.at[pl.ds(base, CHUNK)])
        pltpu.sync_copy(out_v.at[pl.ds(CHUNK, CHUNK)],
                        soft_hbm.at[pl.ds(base, CHUNK)])
        pltpu.sync_copy(out_v.at[pl.ds(2 * CHUNK, CHUNK)],
                        hard_hbm.at[pl.ds(base, CHUNK)])

    @pl.when(q == 31)
    def _():
        pltpu.sync_copy(out_v.at[pl.ds(0, TAIL)],
                        probs_hbm.at[pl.ds(base, TAIL)])
        pltpu.sync_copy(out_v.at[pl.ds(CHUNK, TAIL)],
                        soft_hbm.at[pl.ds(base, TAIL)])
        pltpu.sync_copy(out_v.at[pl.ds(2 * CHUNK, TAIL)],
                        hard_hbm.at[pl.ds(base, TAIL)])


def kernel(x, V_idx, E_idx, num_nodes, num_hyperedges, token_valid, inv_node,
           is_test):
    e = lax.bitcast_convert_type(E_idx.astype(jnp.int32), jnp.float32)
    return _degree_mask(e)
